# baseline (speedup 1.0000x reference)
import functools

import jax
import jax.numpy as jnp
from jax import lax
from jax.experimental import pallas as pl
from jax.experimental.pallas import tpu as pltpu
from jax.experimental.pallas import tpu_sc as plsc

VOC_SIZ = 100000
EMB_DIM = 64
NUM_POS = 2048
B, T = 64, 2048

_NC, _NS, _L = 2, 16, 16
_NW = _NC * _NS
_DPW = EMB_DIM // _NW
_NP = B // 2
_NR = 2


def _emb_body(idx_hbm, table_hbm, pos_hbm, emb_out, tok_out, pos_out,
              row_v, pos_v, idx_bufs, tok_bufs, emb_bufs,
              s_row, s_pos, s_idx, s_tok, s_emb):
    wid = lax.axis_index("s") * _NC + lax.axis_index("c")

    def compute(b0, p, d):
        for o in range(2):
            @plsc.parallel_loop(0, T, step=_L, unroll=8)
            def _grp(t, o=o):
                sl = pl.ds(t, _L)
                iv = idx_bufs[p][o, sl]
                tv = plsc.load_gather(row_v, [iv])
                tok_bufs[p][o, sl] = tv
                emb_bufs[p][o, sl] = tv + pos_v[sl]

        pltpu.async_copy(tok_bufs[p], tok_out.at[pl.ds(b0, 2), d], s_tok[p])
        pltpu.async_copy(emb_bufs[p], emb_out.at[pl.ds(b0, 2), d], s_emb[p])

    def wait_writes(b0, p, d):
        pltpu.make_async_copy(
            tok_bufs[p], tok_out.at[pl.ds(b0, 2), d], s_tok[p]).wait()
        pltpu.make_async_copy(
            emb_bufs[p], emb_out.at[pl.ds(b0, 2), d], s_emb[p]).wait()

    def issue_idx(b0, p):
        pltpu.async_copy(idx_hbm.at[pl.ds(b0, 2)], idx_bufs[p], s_idx[p])

    def wait_idx(b0, p):
        pltpu.make_async_copy(
            idx_hbm.at[pl.ds(b0, 2)], idx_bufs[p], s_idx[p]).wait()

    for u in range(_DPW):
        d = wid + u * _NW
        cp_row = pltpu.async_copy(table_hbm.at[d], row_v, s_row)
        cp_pos = pltpu.async_copy(pos_hbm.at[d], pos_v, s_pos)
        issue_idx(0, 0)
        issue_idx(2, 1)
        cp_pos.wait()
        pltpu.sync_copy(pos_v, pos_out.at[d])
        cp_row.wait()

        for s in range(_NR):
            b0 = 2 * s
            wait_idx(b0, s)
            if u > 0:
                wait_writes(B - 2 * _NR + b0, s, d - _NW)
            compute(b0, s, d)
            issue_idx(b0 + 2 * _NR, s)

        def blk(ss, c, d=d):
            for par in range(_NR):
                b0 = ss * 4 + 2 * par
                wait_idx(b0, par)
                wait_writes(b0 - 2 * _NR, par, d)
                compute(b0, par, d)

                @pl.when(b0 + 2 * _NR < B)
                def _():
                    issue_idx(b0 + 2 * _NR, par)
            return c
        lax.fori_loop(1, _NP // _NR, blk, 0)

    d_last = wid + (_DPW - 1) * _NW
    wait_writes(B - 4, 0, d_last)
    wait_writes(B - 2, 1, d_last)


@jax.jit
def _emb_call(idx2d, table_t, pos_t):
    mesh = plsc.VectorSubcoreMesh(core_axis_name="c", subcore_axis_name="s")
    out_type = (
        jax.ShapeDtypeStruct((B, EMB_DIM, T), jnp.float32),
        jax.ShapeDtypeStruct((B, EMB_DIM, T), jnp.float32),
        jax.ShapeDtypeStruct((EMB_DIM, NUM_POS), jnp.float32),
    )
    scratch = [
        pltpu.VMEM((VOC_SIZ,), jnp.float32),
        pltpu.VMEM((T,), jnp.float32),
        [pltpu.VMEM((2, T), jnp.int32)] * _NR,
        [pltpu.VMEM((2, T), jnp.float32)] * _NR,
        [pltpu.VMEM((2, T), jnp.float32)] * _NR,
        pltpu.SemaphoreType.DMA,
        pltpu.SemaphoreType.DMA,
        [pltpu.SemaphoreType.DMA] * _NR,
        [pltpu.SemaphoreType.DMA] * _NR,
        [pltpu.SemaphoreType.DMA] * _NR,
    ]
    fn = functools.partial(
        pl.kernel, mesh=mesh, out_type=out_type, scratch_types=scratch,
        compiler_params=pltpu.CompilerParams(
            needs_layout_passes=False,
            disable_bounds_checks=True,
            disable_semaphore_checks=True,
        ),
    )(_emb_body)
    return fn(idx2d, table_t, pos_t)


def kernel(batInpTokSeq, token_table, pos_table):
    idx2d = batInpTokSeq.astype(jnp.int32)
    emb, tok, pos = _emb_call(idx2d, token_table.T, pos_table.T)
    return (
        emb.transpose(0, 2, 1),
        tok.transpose(0, 2, 1),
        pos.T,
    )


# speedup vs baseline: 1.0743x; 1.0743x over previous
import functools

import jax
import jax.numpy as jnp
from jax import lax
from jax.experimental import pallas as pl
from jax.experimental.pallas import tpu as pltpu
from jax.experimental.pallas import tpu_sc as plsc

VOC_SIZ = 100000
EMB_DIM = 64
NUM_POS = 2048
B, T = 64, 2048

_NC, _NS, _L = 2, 16, 16
_NW = _NC * _NS
_DPW = EMB_DIM // _NW
_NIB = 4
_NOB = 4


def _emb_body(idx_hbm, table_hbm, pos_hbm, emb_out, tok_out, pos_out,
              row_v, pos_v, idx_bufs, tok_bufs, emb_bufs,
              s_row, s_pos, s_idx, s_tok, s_emb):
    wid = lax.axis_index("s") * _NC + lax.axis_index("c")

    def compute(b, jj, d):
        ob = jj % _NOB

        @plsc.parallel_loop(0, T, step=_L, unroll=8)
        def _grp(t):
            sl = pl.ds(t, _L)
            iv = idx_bufs[jj][sl]
            tv = plsc.load_gather(row_v, [iv])
            tok_bufs[ob][sl] = tv
            emb_bufs[ob][sl] = tv + pos_v[sl]

        pltpu.async_copy(tok_bufs[ob], tok_out.at[b, d], s_tok[ob])
        pltpu.async_copy(emb_bufs[ob], emb_out.at[b, d], s_emb[ob])

    def wait_writes(b, jj, d):
        ob = jj % _NOB
        pltpu.make_async_copy(tok_bufs[ob], tok_out.at[b, d], s_tok[ob]).wait()
        pltpu.make_async_copy(emb_bufs[ob], emb_out.at[b, d], s_emb[ob]).wait()

    for u in range(_DPW):
        d = wid + u * _NW
        cp_row = pltpu.async_copy(table_hbm.at[d], row_v, s_row)
        cp_pos = pltpu.async_copy(pos_hbm.at[d], pos_v, s_pos)
        for j in range(_NIB):
            pltpu.async_copy(idx_hbm.at[j], idx_bufs[j], s_idx[j])
        cp_pos.wait()
        pltpu.sync_copy(pos_v, pos_out.at[d])
        cp_row.wait()

        for jj in range(_NIB):
            b = jj
            pltpu.make_async_copy(idx_hbm.at[b], idx_bufs[jj], s_idx[jj]).wait()
            if b >= _NOB:
                wait_writes(b - _NOB, jj, d)
            elif u > 0:
                wait_writes(B - _NOB + b, jj, d - _NW)
            compute(b, jj, d)
            pltpu.async_copy(idx_hbm.at[b + _NIB], idx_bufs[jj], s_idx[jj])

        def blk(g, c, d=d):
            for jj in range(_NIB):
                b = g * _NIB + jj
                pltpu.make_async_copy(
                    idx_hbm.at[b], idx_bufs[jj], s_idx[jj]).wait()
                wait_writes(b - _NOB, jj, d)
                compute(b, jj, d)

                @pl.when(b + _NIB < B)
                def _():
                    pltpu.async_copy(
                        idx_hbm.at[b + _NIB], idx_bufs[jj], s_idx[jj])
            return c
        lax.fori_loop(1, B // _NIB, blk, 0)

    d_last = wid + (_DPW - 1) * _NW
    for b in range(B - _NOB, B):
        wait_writes(b, b % _NIB, d_last)


@jax.jit
def _emb_call(idx2d, table_t, pos_t):
    mesh = plsc.VectorSubcoreMesh(core_axis_name="c", subcore_axis_name="s")
    out_type = (
        jax.ShapeDtypeStruct((B, EMB_DIM, T), jnp.float32),
        jax.ShapeDtypeStruct((B, EMB_DIM, T), jnp.float32),
        jax.ShapeDtypeStruct((EMB_DIM, NUM_POS), jnp.float32),
    )
    scratch = [
        pltpu.VMEM((VOC_SIZ,), jnp.float32),
        pltpu.VMEM((T,), jnp.float32),
        [pltpu.VMEM((T,), jnp.int32)] * _NIB,
        [pltpu.VMEM((T,), jnp.float32)] * _NOB,
        [pltpu.VMEM((T,), jnp.float32)] * _NOB,
        pltpu.SemaphoreType.DMA,
        pltpu.SemaphoreType.DMA,
        [pltpu.SemaphoreType.DMA] * _NIB,
        [pltpu.SemaphoreType.DMA] * _NOB,
        [pltpu.SemaphoreType.DMA] * _NOB,
    ]
    fn = functools.partial(
        pl.kernel, mesh=mesh, out_type=out_type, scratch_types=scratch,
        compiler_params=pltpu.CompilerParams(
            needs_layout_passes=False,
            disable_bounds_checks=True,
            disable_semaphore_checks=True,
        ),
    )(_emb_body)
    return fn(idx2d, table_t, pos_t)


def kernel(batInpTokSeq, token_table, pos_table):
    idx2d = batInpTokSeq.astype(jnp.int32)
    emb, tok, pos = _emb_call(idx2d, token_table.T, pos_table.T)
    return (
        emb.transpose(0, 2, 1),
        tok.transpose(0, 2, 1),
        pos.T,
    )
